# baseline (speedup 1.0000x reference)
import functools

import jax
import jax.numpy as jnp
from jax import lax
from jax.experimental import pallas as pl
from jax.experimental.pallas import tpu as pltpu
from jax.experimental.pallas import tpu_sc as plsc

B = 1024 * 200
DW = 128
DE = 64
DP = 128
NW = 32
PER_W = B // NW
C = 128
NITER = PER_W // C
NBUF = 5
LA = 2

_mesh = plsc.VectorSubcoreMesh(core_axis_name="c", subcore_axis_name="s")


@functools.partial(
    pl.kernel,
    out_type=jax.ShapeDtypeStruct((B, DP), jnp.float32),
    mesh=_mesh,
    scratch_types=[
        pltpu.VMEM((PER_W,), jnp.int32),
        pltpu.VMEM((NBUF, C, DP), jnp.float32),
        pltpu.SemaphoreType.DMA((NBUF,)),
        pltpu.SemaphoreType.DMA((NBUF,)),
    ],
    compiler_params=pltpu.CompilerParams(use_tc_tiling_on_sc=True),
)
def _sc_gather(idx_hbm, table_hbm, gath_hbm, idx_v, rows_v, gsem, ssem):
    wid = lax.axis_index("s") * 2 + lax.axis_index("c")
    base = wid * PER_W

    def fire_gather(ci, b):
        pltpu.async_copy(
            table_hbm.at[idx_v.at[pl.ds(ci * C, C)]], rows_v.at[b], gsem.at[b])

    def wait_gather(b):
        pltpu.make_async_copy(
            table_hbm.at[idx_v.at[pl.ds(0, C)]], rows_v.at[b], gsem.at[b]).wait()

    def fire_store(ci, b):
        pltpu.async_copy(
            rows_v.at[b], gath_hbm.at[pl.ds(base + ci * C, C), :], ssem.at[b])

    def wait_store(b):
        pltpu.make_async_copy(
            rows_v.at[b], gath_hbm.at[pl.ds(0, C), :], ssem.at[b]).wait()

    def step(ci, b, wait_prev_store, fire_next_gather):
        wait_gather(b)
        if fire_next_gather:
            nb = (b + LA) % NBUF
            if wait_prev_store:
                wait_store(nb)
            fire_gather(ci + LA, nb)
        fire_store(ci, b)

    pltpu.sync_copy(idx_hbm.at[pl.ds(base, PER_W)], idx_v)

    for ci in range(LA):
        fire_gather(ci, ci)

    for ci in range(NBUF):
        step(ci, ci % NBUF, wait_prev_store=(ci + LA >= NBUF),
             fire_next_gather=True)

    def body(k, carry):
        for b in range(NBUF):
            step(k * NBUF + b, b, wait_prev_store=True, fire_next_gather=True)
        return carry

    lax.fori_loop(1, NITER // NBUF - 1, body, 0)

    for ci in range(NITER - NBUF, NITER):
        step(ci, ci % NBUF, wait_prev_store=True,
             fire_next_gather=(ci + LA < NITER))

    for b in range(NBUF):
        wait_store(b)


RB = 2048


def _tc_concat_body(wall_ref, gath_ref, out_ref):
    out_ref[:, 0:DW] = wall_ref[...]
    out_ref[:, DW:DW + DE] = gath_ref[:, 0:DE]


_tc_concat = pl.pallas_call(
    _tc_concat_body,
    out_shape=jax.ShapeDtypeStruct((B, DW + DE), jnp.float32),
    grid=(B // RB,),
    in_specs=[
        pl.BlockSpec((RB, DW), lambda i: (i, 0)),
        pl.BlockSpec((RB, DP), lambda i: (i, 0)),
    ],
    out_specs=pl.BlockSpec((RB, DW + DE), lambda i: (i, 0)),
)


def kernel(wall_distances, keymask, key_embed):
    wall2d = wall_distances.reshape(B, DW)
    idx1d = jnp.maximum(keymask.reshape(B), 0)
    table_pad = jnp.pad(key_embed, ((0, 0), (0, DP - DE)))
    gath = _sc_gather(idx1d, table_pad)
    out = _tc_concat(wall2d, gath)
    return out.reshape(1024, 200, DW + DE)


# speedup vs baseline: 1.1548x; 1.1548x over previous
import functools

import jax
import jax.numpy as jnp
from jax import lax
from jax.experimental import pallas as pl
from jax.experimental.pallas import tpu as pltpu
from jax.experimental.pallas import tpu_sc as plsc

B = 1024 * 200
DW = 128
DE = 64
DP = 128
NW = 32
PER_W = B // NW
C = 128
NITER = PER_W // C
NBUF = 5
LA = 2

_mesh = plsc.VectorSubcoreMesh(core_axis_name="c", subcore_axis_name="s")


@functools.partial(
    pl.kernel,
    out_type=jax.ShapeDtypeStruct((B, DP), jnp.float32),
    mesh=_mesh,
    scratch_types=[
        pltpu.VMEM((PER_W,), jnp.int32),
        pltpu.VMEM((NBUF, C, DP), jnp.float32),
        pltpu.SemaphoreType.DMA((NBUF,)),
        pltpu.SemaphoreType.DMA((NBUF,)),
    ],
    compiler_params=pltpu.CompilerParams(use_tc_tiling_on_sc=True),
)
def _sc_gather(idx_hbm, table_hbm, gath_hbm, idx_v, rows_v, gsem, ssem):
    wid = lax.axis_index("s") * 2 + lax.axis_index("c")
    base = wid * PER_W

    def fire_gather(ci, b):
        pltpu.async_copy(
            table_hbm.at[idx_v.at[pl.ds(ci * C, C)]], rows_v.at[b], gsem.at[b])

    def wait_gather(b):
        pltpu.make_async_copy(
            table_hbm.at[idx_v.at[pl.ds(0, C)]], rows_v.at[b], gsem.at[b]).wait()

    def fire_store(ci, b):
        pltpu.async_copy(
            rows_v.at[b], gath_hbm.at[pl.ds(base + ci * C, C), :], ssem.at[b])

    def wait_store(b):
        pltpu.make_async_copy(
            rows_v.at[b], gath_hbm.at[pl.ds(0, C), :], ssem.at[b]).wait()

    def step(ci, b, wait_prev_store, fire_next_gather):
        wait_gather(b)
        if fire_next_gather:
            nb = (b + LA) % NBUF
            if wait_prev_store:
                wait_store(nb)
            fire_gather(ci + LA, nb)
        fire_store(ci, b)

    pltpu.sync_copy(idx_hbm.at[pl.ds(base, PER_W)], idx_v)

    for ci in range(LA):
        fire_gather(ci, ci)

    for ci in range(NBUF):
        step(ci, ci % NBUF, wait_prev_store=(ci + LA >= NBUF),
             fire_next_gather=True)

    def body(k, carry):
        for b in range(NBUF):
            step(k * NBUF + b, b, wait_prev_store=True, fire_next_gather=True)
        return carry

    lax.fori_loop(1, NITER // NBUF - 1, body, 0)

    for ci in range(NITER - NBUF, NITER):
        step(ci, ci % NBUF, wait_prev_store=True,
             fire_next_gather=(ci + LA < NITER))

    for b in range(NBUF):
        wait_store(b)


RB = 2048


def _tc_concat_body(wall_ref, gath_ref, out_ref):
    out_ref[:, 0:DW] = wall_ref[...]
    out_ref[:, DW:DW + DE] = gath_ref[:, 0:DE]


_tc_concat = pl.pallas_call(
    _tc_concat_body,
    out_shape=jax.ShapeDtypeStruct((B, DW + DE), jnp.float32),
    grid=(B // RB,),
    in_specs=[
        pl.BlockSpec((RB, DW), lambda i: (i, 0)),
        pl.BlockSpec((RB, DP), lambda i: (i, 0)),
    ],
    out_specs=pl.BlockSpec((RB, DW + DE), lambda i: (i, 0)),
)


def kernel(wall_distances, keymask, key_embed):
    wall2d = wall_distances.reshape(B, DW)
    idx1d = jnp.maximum(keymask.reshape(B), 0)
    table_pad = jnp.pad(key_embed, ((0, 0), (0, DP - DE)))
    gath = _sc_gather(idx1d, table_pad)
    out = jnp.concatenate([wall2d, gath[:, :DE]], axis=1)
    return out.reshape(1024, 200, DW + DE)
